# baseline (speedup 1.0000x reference)
import functools
import math

import jax
import jax.numpy as jnp
from jax import lax
from jax.experimental import pallas as pl
from jax.experimental.pallas import tpu as pltpu
from jax.experimental.pallas import tpu_sc as plsc

EMB_DIM = 64
SCALE = math.sqrt(EMB_DIM)

NC = 2
NS = 16
NW = NC * NS
LANES = 16

CHUNK = 128


def _make_kernel(n_chunks):
    mesh = plsc.VectorSubcoreMesh(core_axis_name="c", subcore_axis_name="s")
    per_worker = n_chunks * CHUNK
    total = NW * per_worker

    @functools.partial(
        pl.kernel,
        out_type=jax.ShapeDtypeStruct((total, EMB_DIM), jnp.float32),
        mesh=mesh,
        scratch_types=[
            pltpu.VMEM((n_chunks, CHUNK), jnp.int32),
            pltpu.VMEM((CHUNK, EMB_DIM), jnp.float32),
            pltpu.SemaphoreType.DMA,
        ],
        compiler_params=pltpu.CompilerParams(use_tc_tiling_on_sc=False),
    )
    def gather_scale(table_hbm, tok_hbm, out_hbm, idx_v, buf, sem):
        wid = lax.axis_index("s") * NC + lax.axis_index("c")
        base = wid * per_worker
        pltpu.sync_copy(tok_hbm.at[wid], idx_v)

        def chunk_body(j, _):
            pltpu.async_copy(table_hbm.at[idx_v.at[j]], buf, sem).wait()

            def row_body(r, _):
                for k in range(EMB_DIM // LANES):
                    sl = (r, pl.ds(k * LANES, LANES))
                    buf[sl] = buf[sl] * SCALE
                return 0

            lax.fori_loop(0, CHUNK, row_body, 0, unroll=2)
            pltpu.sync_copy(buf, out_hbm.at[pl.ds(base + j * CHUNK, CHUNK)])
            return 0

        lax.fori_loop(0, n_chunks, chunk_body, 0)

    return gather_scale


@jax.jit
def kernel(tokens, table):
    b, s = tokens.shape
    flat = tokens.reshape(-1).astype(jnp.int32)
    n = flat.shape[0]
    assert n % (NW * CHUNK) == 0
    n_chunks = n // (NW * CHUNK)
    tok3d = flat.reshape(NW, n_chunks, CHUNK)
    out = _make_kernel(n_chunks)(table, tok3d)
    return out.reshape(b, s, EMB_DIM)
